# baseline (speedup 1.0000x reference)
import functools

import jax
import jax.numpy as jnp
from jax import lax
from jax.experimental import pallas as pl
from jax.experimental.pallas import tpu as pltpu
from jax.experimental.pallas import tpu_sc as plsc

N = 10000
NPAD = 10112
F = 256
D = 128
S = 1000
SPAD = 1024

NC = 2
NS = 16
NW = NC * NS
CHUNK = 128

CHUNKS0 = 16
CHUNKS1 = 64


def _mm_body(x_ref, w_ref, o_ref):
    o_ref[...] = jnp.dot(x_ref[...], w_ref[...],
                         preferred_element_type=jnp.float32)


def _feature_matmul(features, W):
    blk = 2000
    return pl.pallas_call(
        _mm_body,
        out_shape=jax.ShapeDtypeStruct((N, D), jnp.float32),
        grid=(N // blk,),
        in_specs=[pl.BlockSpec((blk, F), lambda i: (i, 0)),
                  pl.BlockSpec((F, D), lambda i: (0, 0))],
        out_specs=pl.BlockSpec((blk, D), lambda i: (i, 0)),
    )(features, W)


def _sc_segment_sum(h, src2, dst2, samp, zeros, chunks0, chunks1):
    n_rows = src2.shape[0]
    assert n_rows == NS * (chunks0 + chunks1)
    max_chunks = max(chunks0, chunks1)
    rows_per_tile = NPAD // NS
    sp_per_tile = SPAD // NS

    mesh = plsc.VectorSubcoreMesh(core_axis_name="c", subcore_axis_name="s")

    @functools.partial(
        pl.kernel,
        out_type=(jax.ShapeDtypeStruct((NC * NPAD, D), jnp.float32),
                  jax.ShapeDtypeStruct((NC * SPAD, D), jnp.float32)),
        mesh=mesh,
        scratch_types=[
            pltpu.VMEM((max_chunks, CHUNK), jnp.int32),
            pltpu.VMEM((max_chunks, CHUNK), jnp.int32),
            pltpu.VMEM((CHUNK, D), jnp.float32),
            pltpu.VMEM((CHUNK, D), jnp.float32),
            pltpu.VMEM((sp_per_tile,), jnp.int32),
            pltpu.VMEM_SHARED((NPAD, D), jnp.float32),
            pltpu.SemaphoreType.DMA,
            pltpu.SemaphoreType.DMA,
            pltpu.SemaphoreType.DMA,
            pltpu.SemaphoreType.DMA,
            pltpu.SemaphoreType.DMA,
            pltpu.SemaphoreType.DMA,
        ],
    )
    def seg_kernel(h_hbm, src_hbm, dst_hbm, samp_hbm, zeros_hbm,
                   p_hbm, zs_hbm,
                   src_v, dst_v, rows0, rows1, sidx_v, accum,
                   zsem, gsem0, gsem1, ssem0, ssem1, samsem):
        c = lax.axis_index("c")
        s = lax.axis_index("s")

        rows = (rows0, rows1)
        gsem = (gsem0, gsem1)
        ssem = (ssem0, ssem1)

        zcp = pltpu.async_copy(
            zeros_hbm.at[pl.ds(s * rows_per_tile, rows_per_tile)],
            accum.at[pl.ds(s * rows_per_tile, rows_per_tile)], zsem)

        def g_start(j, b):
            pltpu.async_copy(h_hbm.at[src_v.at[j]], rows[b], gsem[b])

        def g_wait(j, b):
            pltpu.make_async_copy(h_hbm.at[src_v.at[j]], rows[b],
                                  gsem[b]).wait()

        def s_start(j, b):
            pltpu.async_copy(rows[b], accum.at[dst_v.at[j]], ssem[b],
                             add=True)

        def s_wait(j, b):
            pltpu.make_async_copy(rows[b], accum.at[dst_v.at[j]],
                                  ssem[b]).wait()

        def run_core(base_r, n_ch):
            pltpu.sync_copy(src_hbm.at[pl.ds(base_r, n_ch)],
                            src_v.at[pl.ds(0, n_ch)])
            pltpu.sync_copy(dst_hbm.at[pl.ds(base_r, n_ch)],
                            dst_v.at[pl.ds(0, n_ch)])
            zcp.wait()
            plsc.subcore_barrier()

            g_start(0, 0)

            @pl.loop(0, n_ch, step=2)
            def _pipeline(j):
                for b in range(2):
                    jj = j + b
                    g_wait(jj, b)
                    s_start(jj, b)

                    @pl.when(jj >= 1)
                    def _():
                        s_wait(jj - 1, 1 - b)

                    @pl.when(jj + 1 < n_ch)
                    def _():
                        g_start(jj + 1, 1 - b)

            s_wait(n_ch - 1, (n_ch - 1) % 2)

        @pl.when(c == 0)
        def _():
            run_core(s * chunks0, chunks0)

        @pl.when(c == 1)
        def _():
            run_core(NS * chunks0 + s * chunks1, chunks1)

        plsc.subcore_barrier()

        pltpu.sync_copy(
            accum.at[pl.ds(s * rows_per_tile, rows_per_tile)],
            p_hbm.at[pl.ds(c * NPAD + s * rows_per_tile, rows_per_tile)])

        pltpu.sync_copy(samp_hbm.at[pl.ds(s * sp_per_tile, sp_per_tile)],
                        sidx_v)
        srows = rows0.at[pl.ds(0, sp_per_tile)]
        pltpu.async_copy(accum.at[sidx_v], srows, samsem).wait()
        pltpu.sync_copy(
            srows,
            zs_hbm.at[pl.ds(c * SPAD + s * sp_per_tile, sp_per_tile)])

    return seg_kernel(h, src2, dst2, samp, zeros)


def _dec_body(p0_ref, p1_ref, zs0_ref, zs1_ref, zm_ref, ip_ref, dist_ref):
    zm_ref[...] = p0_ref[...] + p1_ref[...]
    z = zs0_ref[...][:S] + zs1_ref[...][:S]
    ip = lax.dot_general(z, z, (((1,), (1,)), ((), ())),
                         preferred_element_type=jnp.float32)
    ip_ref[...] = ip
    sq = jnp.sum(z * z, axis=1)
    dist_ref[...] = sq[:, None] - 2.0 * ip + sq[None, :]


def _decoder(p0, p1, zs0, zs1):
    return pl.pallas_call(
        _dec_body,
        out_shape=(jax.ShapeDtypeStruct((N, D), jnp.float32),
                   jax.ShapeDtypeStruct((S, S), jnp.float32),
                   jax.ShapeDtypeStruct((S, S), jnp.float32)),
    )(p0, p1, zs0, zs1)


def kernel(features, edge_index, sampled_nodes, W):
    src = edge_index[0].astype(jnp.int32)
    dst = edge_index[1].astype(jnp.int32)
    e = src.shape[0]
    epad = NS * (CHUNKS0 + CHUNKS1) * CHUNK
    assert epad >= e
    src_p = jnp.concatenate([src, jnp.zeros((epad - e,), jnp.int32)])
    dst_p = jnp.concatenate([dst, jnp.full((epad - e,), N, jnp.int32)])
    src_p = src_p.reshape(epad // CHUNK, CHUNK)
    dst_p = dst_p.reshape(epad // CHUNK, CHUNK)
    samp = sampled_nodes.astype(jnp.int32)
    samp_p = jnp.concatenate([samp, jnp.zeros((SPAD - S,), jnp.int32)])
    zeros = jnp.zeros((NPAD, D), jnp.float32)

    h = _feature_matmul(features, W)
    p_flat, zs_flat = _sc_segment_sum(h, src_p, dst_p, samp_p, zeros,
                                      CHUNKS0, CHUNKS1)

    p0 = p_flat[:N]
    p1 = p_flat[NPAD:NPAD + N]
    zs0 = zs_flat[:SPAD]
    zs1 = zs_flat[SPAD:]

    z_mean, ip, dist = _decoder(p0, p1, zs0, zs1)
    return z_mean, ip.reshape(-1), dist.reshape(-1)


# speedup vs baseline: 1.1705x; 1.1705x over previous
import functools

import jax
import jax.numpy as jnp
from jax import lax
from jax.experimental import pallas as pl
from jax.experimental.pallas import tpu as pltpu
from jax.experimental.pallas import tpu_sc as plsc

N = 10000
NPAD = 10112
F = 256
D = 128
S = 1000
SPAD = 1024

NC = 2
NS = 16
NW = NC * NS
CHUNK = 128

CHUNKS0 = 64
CHUNKS1 = 16


def _mm_body(x_ref, w_ref, o_ref):
    o_ref[...] = jnp.dot(x_ref[...], w_ref[...],
                         preferred_element_type=jnp.float32)


def _feature_matmul(features, W):
    blk = 2000
    return pl.pallas_call(
        _mm_body,
        out_shape=jax.ShapeDtypeStruct((N, D), jnp.float32),
        grid=(N // blk,),
        in_specs=[pl.BlockSpec((blk, F), lambda i: (i, 0)),
                  pl.BlockSpec((F, D), lambda i: (0, 0))],
        out_specs=pl.BlockSpec((blk, D), lambda i: (i, 0)),
    )(features, W)


def _sc_segment_sum(h, src2, dst2, samp, zeros, chunks0, chunks1):
    n_rows = src2.shape[0]
    assert n_rows == NS * (chunks0 + chunks1)
    max_chunks = max(chunks0, chunks1)
    rows_per_tile = NPAD // NS
    sp_per_tile = SPAD // NS

    mesh = plsc.VectorSubcoreMesh(core_axis_name="c", subcore_axis_name="s")

    @functools.partial(
        pl.kernel,
        out_type=(jax.ShapeDtypeStruct((NC * NPAD, D), jnp.float32),
                  jax.ShapeDtypeStruct((NC * SPAD, D), jnp.float32)),
        mesh=mesh,
        scratch_types=[
            pltpu.VMEM((max_chunks, CHUNK), jnp.int32),
            pltpu.VMEM((max_chunks, CHUNK), jnp.int32),
            pltpu.VMEM((CHUNK, D), jnp.float32),
            pltpu.VMEM((CHUNK, D), jnp.float32),
            pltpu.VMEM((sp_per_tile,), jnp.int32),
            pltpu.VMEM_SHARED((NPAD, D), jnp.float32),
            pltpu.SemaphoreType.DMA,
            pltpu.SemaphoreType.DMA,
            pltpu.SemaphoreType.DMA,
            pltpu.SemaphoreType.DMA,
            pltpu.SemaphoreType.DMA,
            pltpu.SemaphoreType.DMA,
        ],
    )
    def seg_kernel(h_hbm, src_hbm, dst_hbm, samp_hbm, zeros_hbm,
                   p_hbm, zs_hbm,
                   src_v, dst_v, rows0, rows1, sidx_v, accum,
                   zsem, gsem0, gsem1, ssem0, ssem1, samsem):
        c = lax.axis_index("c")
        s = lax.axis_index("s")

        rows = (rows0, rows1)
        gsem = (gsem0, gsem1)
        ssem = (ssem0, ssem1)

        zcp = pltpu.async_copy(
            zeros_hbm.at[pl.ds(s * rows_per_tile, rows_per_tile)],
            accum.at[pl.ds(s * rows_per_tile, rows_per_tile)], zsem)

        def g_start(j, b):
            pltpu.async_copy(h_hbm.at[src_v.at[j]], rows[b], gsem[b])

        def g_wait(j, b):
            pltpu.make_async_copy(h_hbm.at[src_v.at[j]], rows[b],
                                  gsem[b]).wait()

        def s_start(j, b):
            pltpu.async_copy(rows[b], accum.at[dst_v.at[j]], ssem[b],
                             add=True)

        def s_wait(j, b):
            pltpu.make_async_copy(rows[b], accum.at[dst_v.at[j]],
                                  ssem[b]).wait()

        def run_core(base_r, n_ch):
            pltpu.sync_copy(src_hbm.at[pl.ds(base_r, n_ch)],
                            src_v.at[pl.ds(0, n_ch)])
            pltpu.sync_copy(dst_hbm.at[pl.ds(base_r, n_ch)],
                            dst_v.at[pl.ds(0, n_ch)])
            zcp.wait()
            plsc.subcore_barrier()

            g_start(0, 0)

            @pl.loop(0, n_ch, step=2)
            def _pipeline(j):
                for b in range(2):
                    jj = j + b
                    g_wait(jj, b)
                    s_start(jj, b)

                    @pl.when(jj >= 1)
                    def _():
                        s_wait(jj - 1, 1 - b)

                    @pl.when(jj + 1 < n_ch)
                    def _():
                        g_start(jj + 1, 1 - b)

            s_wait(n_ch - 1, (n_ch - 1) % 2)

        @pl.when(c == 0)
        def _():
            run_core(s * chunks0, chunks0)

        @pl.when(c == 1)
        def _():
            run_core(NS * chunks0 + s * chunks1, chunks1)

        plsc.subcore_barrier()

        pltpu.sync_copy(
            accum.at[pl.ds(s * rows_per_tile, rows_per_tile)],
            p_hbm.at[pl.ds(c * NPAD + s * rows_per_tile, rows_per_tile)])

        pltpu.sync_copy(samp_hbm.at[pl.ds(s * sp_per_tile, sp_per_tile)],
                        sidx_v)
        srows = rows0.at[pl.ds(0, sp_per_tile)]
        pltpu.async_copy(accum.at[sidx_v], srows, samsem).wait()
        pltpu.sync_copy(
            srows,
            zs_hbm.at[pl.ds(c * SPAD + s * sp_per_tile, sp_per_tile)])

    return seg_kernel(h, src2, dst2, samp, zeros)


def _dec_body(p0_ref, p1_ref, zs0_ref, zs1_ref, zm_ref, ip_ref, dist_ref):
    zm_ref[...] = p0_ref[...] + p1_ref[...]
    z = zs0_ref[...][:S] + zs1_ref[...][:S]
    ip = lax.dot_general(z, z, (((1,), (1,)), ((), ())),
                         preferred_element_type=jnp.float32)
    ip_ref[...] = ip
    sq = jnp.sum(z * z, axis=1)
    dist_ref[...] = sq[:, None] - 2.0 * ip + sq[None, :]


def _decoder(p0, p1, zs0, zs1):
    return pl.pallas_call(
        _dec_body,
        out_shape=(jax.ShapeDtypeStruct((N, D), jnp.float32),
                   jax.ShapeDtypeStruct((S, S), jnp.float32),
                   jax.ShapeDtypeStruct((S, S), jnp.float32)),
    )(p0, p1, zs0, zs1)


def kernel(features, edge_index, sampled_nodes, W):
    src = edge_index[0].astype(jnp.int32)
    dst = edge_index[1].astype(jnp.int32)
    e = src.shape[0]
    epad = NS * (CHUNKS0 + CHUNKS1) * CHUNK
    assert epad >= e
    src_p = jnp.concatenate([src, jnp.zeros((epad - e,), jnp.int32)])
    dst_p = jnp.concatenate([dst, jnp.full((epad - e,), N, jnp.int32)])
    src_p = src_p.reshape(epad // CHUNK, CHUNK)
    dst_p = dst_p.reshape(epad // CHUNK, CHUNK)
    samp = sampled_nodes.astype(jnp.int32)
    samp_p = jnp.concatenate([samp, jnp.zeros((SPAD - S,), jnp.int32)])
    zeros = jnp.zeros((NPAD, D), jnp.float32)

    h = _feature_matmul(features, W)
    p_flat, zs_flat = _sc_segment_sum(h, src_p, dst_p, samp_p, zeros,
                                      CHUNKS0, CHUNKS1)

    p0 = p_flat[:N]
    p1 = p_flat[NPAD:NPAD + N]
    zs0 = zs_flat[:SPAD]
    zs1 = zs_flat[SPAD:]

    z_mean, ip, dist = _decoder(p0, p1, zs0, zs1)
    return z_mean, ip.reshape(-1), dist.reshape(-1)


# speedup vs baseline: 2.7529x; 2.3518x over previous
import functools

import jax
import jax.numpy as jnp
from jax import lax
from jax.experimental import pallas as pl
from jax.experimental.pallas import tpu as pltpu
from jax.experimental.pallas import tpu_sc as plsc

N = 10000
NPAD = 10112
F = 256
D = 128
S = 1000
SPAD = 1024

NC = 2
NS = 16
NW = NC * NS
CHUNK = 125
CT = 40
ZROWS = 120


def _mm_body(x_ref, w_ref, o_ref):
    o_ref[...] = jnp.dot(x_ref[...], w_ref[...],
                         preferred_element_type=jnp.float32)


def _feature_matmul(features, W):
    blk = 2000
    return pl.pallas_call(
        _mm_body,
        out_shape=jax.ShapeDtypeStruct((N, D), jnp.float32),
        grid=(N // blk,),
        in_specs=[pl.BlockSpec((blk, F), lambda i: (i, 0)),
                  pl.BlockSpec((F, D), lambda i: (0, 0))],
        out_specs=pl.BlockSpec((blk, D), lambda i: (i, 0)),
    )(features, W)


def _sc_segment_sum(h, idx2, samp):
    n_half = idx2.shape[0] // 2
    assert n_half == NW * CT
    rows_per_tile = NPAD // NS
    sp_per_tile = SPAD // NS

    mesh = plsc.VectorSubcoreMesh(core_axis_name="c", subcore_axis_name="s")

    @functools.partial(
        pl.kernel,
        out_type=(jax.ShapeDtypeStruct((NC * NPAD, D), jnp.float32),
                  jax.ShapeDtypeStruct((NC * SPAD, D), jnp.float32)),
        mesh=mesh,
        scratch_types=[
            pltpu.VMEM((CT, CHUNK), jnp.int32),
            pltpu.VMEM((CT, CHUNK), jnp.int32),
            pltpu.VMEM((CHUNK, D), jnp.float32),
            pltpu.VMEM((CHUNK, D), jnp.float32),
            pltpu.VMEM((sp_per_tile,), jnp.int32),
            pltpu.VMEM_SHARED((NPAD, D), jnp.float32),
            pltpu.SemaphoreType.DMA,
            pltpu.SemaphoreType.DMA,
            pltpu.SemaphoreType.DMA,
            pltpu.SemaphoreType.DMA,
            pltpu.SemaphoreType.DMA,
        ],
    )
    def seg_kernel(h_hbm, idx_hbm, samp_hbm,
                   p_hbm, zs_hbm,
                   src_v, dst_v, rows0, rows1, sidx_v, accum,
                   gsem0, gsem1, ssem0, ssem1, samsem):
        c = lax.axis_index("c")
        s = lax.axis_index("s")
        wid = s * NC + c

        rows = (rows0, rows1)
        gsem = (gsem0, gsem1)
        ssem = (ssem0, ssem1)

        z16 = jnp.zeros((16,), jnp.float32)

        @pl.loop(0, ZROWS)
        def _zrow(r):
            for k in range(D // 16):
                rows0[r, pl.ds(k * 16, 16)] = z16

        nfull, rem = divmod(rows_per_tile, ZROWS)
        for j in range(nfull):
            pltpu.sync_copy(
                rows0.at[pl.ds(0, ZROWS)],
                accum.at[pl.ds(s * rows_per_tile + j * ZROWS, ZROWS)])
        if rem:
            pltpu.sync_copy(
                rows0.at[pl.ds(0, rem)],
                accum.at[pl.ds(s * rows_per_tile + nfull * ZROWS, rem)])

        pltpu.sync_copy(idx_hbm.at[pl.ds(wid * CT, CT)], src_v)
        pltpu.sync_copy(idx_hbm.at[pl.ds(n_half + wid * CT, CT)], dst_v)

        plsc.subcore_barrier()

        def g_start(j, b):
            pltpu.async_copy(h_hbm.at[src_v.at[j]], rows[b], gsem[b])

        def g_wait(j, b):
            pltpu.make_async_copy(h_hbm.at[src_v.at[j]], rows[b],
                                  gsem[b]).wait()

        def s_start(j, b):
            pltpu.async_copy(rows[b], accum.at[dst_v.at[j]], ssem[b],
                             add=True)

        def s_wait(j, b):
            pltpu.make_async_copy(rows[b], accum.at[dst_v.at[j]],
                                  ssem[b]).wait()

        g_start(0, 0)

        @pl.loop(0, CT, step=2)
        def _pipeline(j):
            for b in range(2):
                jj = j + b
                g_wait(jj, b)
                s_start(jj, b)

                @pl.when(jj >= 1)
                def _():
                    s_wait(jj - 1, 1 - b)

                @pl.when(jj + 1 < CT)
                def _():
                    g_start(jj + 1, 1 - b)

        s_wait(CT - 1, (CT - 1) % 2)
        plsc.subcore_barrier()

        pltpu.sync_copy(
            accum.at[pl.ds(s * rows_per_tile, rows_per_tile)],
            p_hbm.at[pl.ds(c * NPAD + s * rows_per_tile, rows_per_tile)])

        pltpu.sync_copy(samp_hbm.at[pl.ds(s * sp_per_tile, sp_per_tile)],
                        sidx_v)
        srows = rows0.at[pl.ds(0, sp_per_tile)]
        pltpu.async_copy(accum.at[sidx_v], srows, samsem).wait()
        pltpu.sync_copy(
            srows,
            zs_hbm.at[pl.ds(c * SPAD + s * sp_per_tile, sp_per_tile)])

    return seg_kernel(h, idx2, samp)


def _dec_body(p_ref, zs_ref, zm_ref, ip_ref, dist_ref):
    p = p_ref[...]
    zm_ref[...] = p[:N] + p[NPAD:NPAD + N]
    zs = zs_ref[...]
    z = zs[:S] + zs[SPAD:SPAD + S]
    ip = lax.dot_general(z, z, (((1,), (1,)), ((), ())),
                         preferred_element_type=jnp.float32)
    ip_ref[...] = ip
    sq = jnp.sum(z * z, axis=1)
    dist_ref[...] = sq[:, None] - 2.0 * ip + sq[None, :]


def _decoder(p_flat, zs_flat):
    return pl.pallas_call(
        _dec_body,
        out_shape=(jax.ShapeDtypeStruct((N, D), jnp.float32),
                   jax.ShapeDtypeStruct((S, S), jnp.float32),
                   jax.ShapeDtypeStruct((S, S), jnp.float32)),
    )(p_flat, zs_flat)


def kernel(features, edge_index, sampled_nodes, W):
    e = edge_index.shape[1]
    assert e == NW * CT * CHUNK
    idx2 = edge_index.astype(jnp.int32).reshape(2 * (e // CHUNK), CHUNK)
    samp = sampled_nodes.astype(jnp.int32)
    samp_p = jnp.concatenate([samp, jnp.zeros((SPAD - S,), jnp.int32)])

    h = _feature_matmul(features, W)
    p_flat, zs_flat = _sc_segment_sum(h, idx2, samp_p)
    z_mean, ip, dist = _decoder(p_flat, zs_flat)
    return z_mean, ip.reshape(-1), dist.reshape(-1)
